# baseline (speedup 1.0000x reference)
import jax
import jax.numpy as jnp
from jax.experimental import pallas as pl


_N, _D = 8192, 2048
_BLK = 512


def _body(q_ref, x_ref, t_ref, out_ref):
    i = pl.program_id(0)

    @pl.when(i == 0)
    def _init():
        out_ref[...] = jnp.zeros_like(out_ref)

    x = x_ref[...]
    t = t_ref[...]
    qb = q_ref[...]
    d = x - t
    w = jnp.where(qb == 1, t, jnp.where(qb == 2, 1.0, 0.0).astype(jnp.float32))
    g = w * d
    out_ref[...] += jnp.sum(g * g).reshape(1, 1)


def kernel(input, target, q):
    q2 = q[:, None]
    grid = _N // _BLK
    total = pl.pallas_call(
        _body,
        grid=(grid,),
        in_specs=[
            pl.BlockSpec((_BLK, 1), lambda i: (i, 0)),
            pl.BlockSpec((_BLK, _D), lambda i: (i, 0)),
            pl.BlockSpec((_BLK, _D), lambda i: (i, 0)),
        ],
        out_specs=pl.BlockSpec((1, 1), lambda i: (0, 0)),
        out_shape=jax.ShapeDtypeStruct((1, 1), jnp.float32),
    )(q2, input, target)
    return total[0, 0] / (_N * _D)
